# baseline (speedup 1.0000x reference)
import functools

import jax
import jax.numpy as jnp
from jax.experimental import pallas as pl
from jax.experimental.pallas import tpu as pltpu

_MM = (((1,), (0,)), ((), ()))


def _dhcf_kernel(adj_ref, ut_ref, i_ref, hu_ref, hit_ref,
                 su_ref, ti_ref, tu_ref, tit_ref, shi_ref,
                 *, bu, bi_sz, nbu, nbi, leaky):
    p = pl.program_id(0)
    bi = pl.program_id(1)
    bj = pl.program_id(2)

    first = (bi == 0) & (bj == 0)
    last = (bi == nbu - 1) & (bj == nbi - 1)

    @pl.when((p == 0) & first)
    def _init():
        su_ref[...] = jnp.zeros_like(su_ref)
        shi_ref[...] = jnp.zeros_like(shi_ref)

    a = adj_ref[...]

    @pl.when(p == 0)
    def _phase0():
        ut_blk = ut_ref[:, pl.ds(bi * bu, bu)]
        i_blk = i_ref[...]
        su_ref[:, pl.ds(bj * bi_sz, bi_sz)] += jax.lax.dot_general(
            ut_blk, a, _MM, preferred_element_type=jnp.float32)
        contrib = jax.lax.dot_general(
            a, i_blk, _MM, preferred_element_type=jnp.float32)

        @pl.when(bj == 0)
        def _():
            ti_ref[pl.ds(bi * bu, bu), :] = contrib

        @pl.when(bj != 0)
        def _():
            ti_ref[pl.ds(bi * bu, bu), :] += contrib

    @pl.when((p == 1) & first)
    def _mid():
        tu_ref[...] = su_ref[...].T
        tit_ref[...] = ti_ref[...].T

    @pl.when(p == 1)
    def _phase1():
        tu_blk = tu_ref[pl.ds(bj * bi_sz, bi_sz), :]
        tit_blk = tit_ref[:, pl.ds(bi * bu, bu)]
        contrib = jax.lax.dot_general(
            a, tu_blk, _MM, preferred_element_type=jnp.float32)

        @pl.when(bj == 0)
        def _():
            hu_ref[...] = contrib

        @pl.when(bj != 0)
        def _():
            hu_ref[...] += contrib

        shi_ref[:, pl.ds(bj * bi_sz, bi_sz)] += jax.lax.dot_general(
            tit_blk, a, _MM, preferred_element_type=jnp.float32)

        @pl.when(bj == nbi - 1)
        def _act_u():
            hu = hu_ref[...]
            hu_ref[...] = jnp.where(hu >= 0, hu, leaky * hu)

    @pl.when((p == 1) & last)
    def _act_i():
        hi = shi_ref[...]
        hit_ref[...] = jnp.where(hi >= 0, hi, leaky * hi)


@jax.jit
def kernel(adj, user_emb, item_emb):
    n_users, n_items = adj.shape
    hd = user_emb.shape[1]
    bu = min(n_users, 512)
    bi_sz = min(n_items, 512)
    nbu = n_users // bu
    nbi = n_items // bi_sz

    body = functools.partial(_dhcf_kernel, bu=bu, bi_sz=bi_sz,
                             nbu=nbu, nbi=nbi, leaky=0.5)
    h_u, h_i_t = pl.pallas_call(
        body,
        grid=(2, nbu, nbi),
        in_specs=[
            pl.BlockSpec((bu, bi_sz), lambda p, i, j: (i, j)),
            pl.BlockSpec((hd, n_users), lambda p, i, j: (0, 0)),
            pl.BlockSpec((bi_sz, hd), lambda p, i, j: (j, 0)),
        ],
        out_specs=[
            pl.BlockSpec((bu, hd), lambda p, i, j: (i, 0)),
            pl.BlockSpec((hd, n_items), lambda p, i, j: (0, 0)),
        ],
        out_shape=[
            jax.ShapeDtypeStruct((n_users, hd), jnp.float32),
            jax.ShapeDtypeStruct((hd, n_items), jnp.float32),
        ],
        scratch_shapes=[
            pltpu.VMEM((hd, n_items), jnp.float32),
            pltpu.VMEM((n_users, hd), jnp.float32),
            pltpu.VMEM((n_items, hd), jnp.float32),
            pltpu.VMEM((hd, n_users), jnp.float32),
            pltpu.VMEM((hd, n_items), jnp.float32),
        ],
    )(adj, user_emb.T, item_emb)

    user_all = jnp.concatenate([user_emb, h_u, h_u], axis=1)
    h_i = h_i_t.T
    item_all = jnp.concatenate([item_emb, h_i, h_i], axis=1)
    return (user_all, item_all)


# speedup vs baseline: 1.9097x; 1.9097x over previous
import functools

import jax
import jax.numpy as jnp
from jax.experimental import pallas as pl
from jax.experimental.pallas import tpu as pltpu

_MM = (((1,), (0,)), ((), ()))


def _dhcf_kernel(adj_ref, ut_ref, i_ref, hu_ref, hit_ref,
                 su_ref, ti_ref, tu_ref, tit_ref, shi_ref,
                 *, bu, bi_sz, nbu, nbi, leaky):
    p = pl.program_id(0)
    bi = pl.program_id(1)
    bj = pl.program_id(2)

    first = (bi == 0) & (bj == 0)
    last = (bi == nbu - 1) & (bj == nbi - 1)

    @pl.when((p == 0) & first)
    def _init():
        su_ref[...] = jnp.zeros_like(su_ref)
        shi_ref[...] = jnp.zeros_like(shi_ref)

    a = adj_ref[...]

    @pl.when(p == 0)
    def _phase0():
        ut_blk = ut_ref[:, pl.ds(bi * bu, bu)]
        i_blk = i_ref[...]
        su_ref[:, pl.ds(bj * bi_sz, bi_sz)] += jax.lax.dot_general(
            ut_blk, a, _MM, preferred_element_type=jnp.float32)
        contrib = jax.lax.dot_general(
            a, i_blk, _MM, preferred_element_type=jnp.float32)

        @pl.when(bj == 0)
        def _():
            ti_ref[pl.ds(bi * bu, bu), :] = contrib

        @pl.when(bj != 0)
        def _():
            ti_ref[pl.ds(bi * bu, bu), :] += contrib

    @pl.when((p == 1) & first)
    def _mid():
        tu_ref[...] = su_ref[...].T
        tit_ref[...] = ti_ref[...].T

    @pl.when(p == 1)
    def _phase1():
        tu_blk = tu_ref[pl.ds(bj * bi_sz, bi_sz), :]
        tit_blk = tit_ref[:, pl.ds(bi * bu, bu)]
        contrib = jax.lax.dot_general(
            a, tu_blk, _MM, preferred_element_type=jnp.float32)

        @pl.when(bj == 0)
        def _():
            hu_ref[...] = contrib

        @pl.when(bj != 0)
        def _():
            hu_ref[...] += contrib

        shi_ref[:, pl.ds(bj * bi_sz, bi_sz)] += jax.lax.dot_general(
            tit_blk, a, _MM, preferred_element_type=jnp.float32)

        @pl.when(bj == nbi - 1)
        def _act_u():
            hu = hu_ref[...]
            hu_ref[...] = jnp.where(hu >= 0, hu, leaky * hu)

    @pl.when((p == 1) & last)
    def _act_i():
        hi = shi_ref[...]
        hit_ref[...] = jnp.where(hi >= 0, hi, leaky * hi)


@jax.jit
def kernel(adj, user_emb, item_emb):
    n_users, n_items = adj.shape
    hd = user_emb.shape[1]
    bu = min(n_users, 1024)
    bi_sz = min(n_items, 1024)
    nbu = n_users // bu
    nbi = n_items // bi_sz

    body = functools.partial(_dhcf_kernel, bu=bu, bi_sz=bi_sz,
                             nbu=nbu, nbi=nbi, leaky=0.5)
    h_u, h_i_t = pl.pallas_call(
        body,
        grid=(2, nbu, nbi),
        in_specs=[
            pl.BlockSpec((bu, bi_sz), lambda p, i, j: (i, j)),
            pl.BlockSpec((hd, n_users), lambda p, i, j: (0, 0)),
            pl.BlockSpec((bi_sz, hd), lambda p, i, j: (j, 0)),
        ],
        out_specs=[
            pl.BlockSpec((bu, hd), lambda p, i, j: (i, 0)),
            pl.BlockSpec((hd, n_items), lambda p, i, j: (0, 0)),
        ],
        out_shape=[
            jax.ShapeDtypeStruct((n_users, hd), jnp.float32),
            jax.ShapeDtypeStruct((hd, n_items), jnp.float32),
        ],
        scratch_shapes=[
            pltpu.VMEM((hd, n_items), jnp.float32),
            pltpu.VMEM((n_users, hd), jnp.float32),
            pltpu.VMEM((n_items, hd), jnp.float32),
            pltpu.VMEM((hd, n_users), jnp.float32),
            pltpu.VMEM((hd, n_items), jnp.float32),
        ],
    )(adj, user_emb.T, item_emb)

    user_all = jnp.concatenate([user_emb, h_u, h_u], axis=1)
    h_i = h_i_t.T
    item_all = jnp.concatenate([item_emb, h_i, h_i], axis=1)
    return (user_all, item_all)


# speedup vs baseline: 1.9497x; 1.0210x over previous
import functools

import jax
import jax.numpy as jnp
from jax.experimental import pallas as pl
from jax.experimental.pallas import tpu as pltpu

_MM = (((1,), (0,)), ((), ()))


def _dhcf_kernel(adj_ref, ut_ref, i_ref, hu_ref, hit_ref,
                 su_ref, ti_ref, tu_ref, tit_ref, shi_ref,
                 *, bu, bi_sz, nbu, nbi, leaky):
    p = pl.program_id(0)
    bi = pl.program_id(1)
    bj = pl.program_id(2)

    first = (bi == 0) & (bj == 0)
    last = (bi == nbu - 1) & (bj == nbi - 1)

    @pl.when((p == 0) & first)
    def _init():
        su_ref[...] = jnp.zeros_like(su_ref)
        shi_ref[...] = jnp.zeros_like(shi_ref)

    a = adj_ref[...].astype(jnp.bfloat16)

    @pl.when(p == 0)
    def _phase0():
        ut_blk = ut_ref[:, pl.ds(bi * bu, bu)].astype(jnp.bfloat16)
        i_blk = i_ref[...].astype(jnp.bfloat16)
        su_ref[:, pl.ds(bj * bi_sz, bi_sz)] += jax.lax.dot_general(
            ut_blk, a, _MM, preferred_element_type=jnp.float32)
        contrib = jax.lax.dot_general(
            a, i_blk, _MM, preferred_element_type=jnp.float32)

        @pl.when(bj == 0)
        def _():
            ti_ref[pl.ds(bi * bu, bu), :] = contrib

        @pl.when(bj != 0)
        def _():
            ti_ref[pl.ds(bi * bu, bu), :] += contrib

    @pl.when((p == 1) & first)
    def _mid():
        tu_ref[...] = su_ref[...].T
        tit_ref[...] = ti_ref[...].T

    @pl.when(p == 1)
    def _phase1():
        tu_blk = tu_ref[pl.ds(bj * bi_sz, bi_sz), :].astype(jnp.bfloat16)
        tit_blk = tit_ref[:, pl.ds(bi * bu, bu)].astype(jnp.bfloat16)
        contrib = jax.lax.dot_general(
            a, tu_blk, _MM, preferred_element_type=jnp.float32)

        @pl.when(bj == 0)
        def _():
            hu_ref[...] = contrib

        @pl.when(bj != 0)
        def _():
            hu_ref[...] += contrib

        shi_ref[:, pl.ds(bj * bi_sz, bi_sz)] += jax.lax.dot_general(
            tit_blk, a, _MM, preferred_element_type=jnp.float32)

        @pl.when(bj == nbi - 1)
        def _act_u():
            hu = hu_ref[...]
            hu_ref[...] = jnp.where(hu >= 0, hu, leaky * hu)

    @pl.when((p == 1) & last)
    def _act_i():
        hi = shi_ref[...]
        hit_ref[...] = jnp.where(hi >= 0, hi, leaky * hi)


@jax.jit
def kernel(adj, user_emb, item_emb):
    n_users, n_items = adj.shape
    hd = user_emb.shape[1]
    bu = min(n_users, 1024)
    bi_sz = min(n_items, 1024)
    nbu = n_users // bu
    nbi = n_items // bi_sz

    body = functools.partial(_dhcf_kernel, bu=bu, bi_sz=bi_sz,
                             nbu=nbu, nbi=nbi, leaky=0.5)
    h_u, h_i_t = pl.pallas_call(
        body,
        grid=(2, nbu, nbi),
        in_specs=[
            pl.BlockSpec((bu, bi_sz), lambda p, i, j: (i, j)),
            pl.BlockSpec((hd, n_users), lambda p, i, j: (0, 0)),
            pl.BlockSpec((bi_sz, hd), lambda p, i, j: (j, 0)),
        ],
        out_specs=[
            pl.BlockSpec((bu, hd), lambda p, i, j: (i, 0)),
            pl.BlockSpec((hd, n_items), lambda p, i, j: (0, 0)),
        ],
        out_shape=[
            jax.ShapeDtypeStruct((n_users, hd), jnp.float32),
            jax.ShapeDtypeStruct((hd, n_items), jnp.float32),
        ],
        scratch_shapes=[
            pltpu.VMEM((hd, n_items), jnp.float32),
            pltpu.VMEM((n_users, hd), jnp.float32),
            pltpu.VMEM((n_items, hd), jnp.float32),
            pltpu.VMEM((hd, n_users), jnp.float32),
            pltpu.VMEM((hd, n_items), jnp.float32),
        ],
    )(adj, user_emb.T, item_emb)

    user_all = jnp.concatenate([user_emb, h_u, h_u], axis=1)
    h_i = h_i_t.T
    item_all = jnp.concatenate([item_emb, h_i, h_i], axis=1)
    return (user_all, item_all)


# speedup vs baseline: 2.5728x; 1.3196x over previous
import functools

import jax
import jax.numpy as jnp
from jax.experimental import pallas as pl
from jax.experimental.pallas import tpu as pltpu

_MM = (((1,), (0,)), ((), ()))
_CH = 512


def _dhcf_kernel(adj_ref, ut_ref, i_ref, hu_ref, hit_ref,
                 ab_ref, su_ref, tit_ref, shi_ref,
                 *, bu, bi_sz, nbu, nbi, leaky):
    p = pl.program_id(0)
    bi = pl.program_id(1)
    bj = pl.program_id(2)

    first = (bi == 0) & (bj == 0)
    last = (bi == nbu - 1) & (bj == nbi - 1)
    nch = bu // _CH

    @pl.when((p == 0) & first)
    def _init():
        su_ref[...] = jnp.zeros_like(su_ref)
        tit_ref[...] = jnp.zeros_like(tit_ref)
        shi_ref[...] = jnp.zeros_like(shi_ref)

    for k in range(nch):
        sl = pl.ds(k * _CH, _CH)
        ab_ref[sl, :] = adj_ref[sl, :].astype(jnp.bfloat16)

    @pl.when(p == 0)
    def _phase0():
        for k in range(nch):
            sl = pl.ds(k * _CH, _CH)
            su_ref[:, pl.ds(bj * bi_sz, bi_sz)] += jax.lax.dot_general(
                ut_ref[:, pl.ds(bi * bu + k * _CH, _CH)].astype(jnp.bfloat16),
                ab_ref[sl, :], _MM, preferred_element_type=jnp.float32)
        contrib = jax.lax.dot_general(
            ab_ref[:, pl.ds(0, _CH)],
            i_ref[pl.ds(0, _CH), :].astype(jnp.bfloat16),
            _MM, preferred_element_type=jnp.float32)
        for k in range(1, bi_sz // _CH):
            contrib += jax.lax.dot_general(
                ab_ref[:, pl.ds(k * _CH, _CH)],
                i_ref[pl.ds(k * _CH, _CH), :].astype(jnp.bfloat16),
                _MM, preferred_element_type=jnp.float32)
        tit_ref[:, pl.ds(bi * bu, bu)] += contrib.T

    @pl.when(p == 1)
    def _phase1():
        hu = jax.lax.dot_general(
            ab_ref[:, pl.ds(0, _CH)],
            su_ref[:, pl.ds(bj * bi_sz, _CH)].astype(jnp.bfloat16).T,
            _MM, preferred_element_type=jnp.float32)
        for k in range(1, bi_sz // _CH):
            hu += jax.lax.dot_general(
                ab_ref[:, pl.ds(k * _CH, _CH)],
                su_ref[:, pl.ds(bj * bi_sz + k * _CH, _CH)]
                .astype(jnp.bfloat16).T,
                _MM, preferred_element_type=jnp.float32)

        @pl.when(bj == 0)
        def _():
            hu_ref[...] = hu

        @pl.when(bj != 0)
        def _():
            hu_ref[...] += hu

        for k in range(nch):
            sl = pl.ds(k * _CH, _CH)
            shi_ref[:, pl.ds(bj * bi_sz, bi_sz)] += jax.lax.dot_general(
                tit_ref[:, pl.ds(bi * bu + k * _CH, _CH)].astype(jnp.bfloat16),
                ab_ref[sl, :], _MM, preferred_element_type=jnp.float32)

        @pl.when(bj == nbi - 1)
        def _act_u():
            huv = hu_ref[...]
            hu_ref[...] = jnp.where(huv >= 0, huv, leaky * huv)

    @pl.when((p == 1) & last)
    def _act_i():
        hi = shi_ref[...]
        hit_ref[...] = jnp.where(hi >= 0, hi, leaky * hi)


@jax.jit
def kernel(adj, user_emb, item_emb):
    n_users, n_items = adj.shape
    hd = user_emb.shape[1]
    bu = min(n_users, 2048)
    bi_sz = min(n_items, 2048)
    nbu = n_users // bu
    nbi = n_items // bi_sz

    body = functools.partial(_dhcf_kernel, bu=bu, bi_sz=bi_sz,
                             nbu=nbu, nbi=nbi, leaky=0.5)
    h_u, h_i_t = pl.pallas_call(
        body,
        grid=(2, nbu, nbi),
        in_specs=[
            pl.BlockSpec((bu, bi_sz), lambda p, i, j: (i, j)),
            pl.BlockSpec((hd, n_users), lambda p, i, j: (0, 0)),
            pl.BlockSpec((bi_sz, hd), lambda p, i, j: (j, 0)),
        ],
        out_specs=[
            pl.BlockSpec((bu, hd), lambda p, i, j: (i, 0)),
            pl.BlockSpec((hd, n_items), lambda p, i, j: (0, 0)),
        ],
        out_shape=[
            jax.ShapeDtypeStruct((n_users, hd), jnp.float32),
            jax.ShapeDtypeStruct((hd, n_items), jnp.float32),
        ],
        scratch_shapes=[
            pltpu.VMEM((bu, bi_sz), jnp.bfloat16),
            pltpu.VMEM((hd, n_items), jnp.float32),
            pltpu.VMEM((hd, n_users), jnp.float32),
            pltpu.VMEM((hd, n_items), jnp.float32),
        ],
    )(adj, user_emb.T, item_emb)

    user_all = jnp.concatenate([user_emb, h_u, h_u], axis=1)
    h_i = h_i_t.T
    item_all = jnp.concatenate([item_emb, h_i, h_i], axis=1)
    return (user_all, item_all)


# speedup vs baseline: 2.7327x; 1.0621x over previous
import functools

import jax
import jax.numpy as jnp
from jax.experimental import pallas as pl
from jax.experimental.pallas import tpu as pltpu

_MM = (((1,), (0,)), ((), ()))
_CH = 512


def _dhcf_kernel(adj_ref, ut_ref, i_ref, hu_ref, hit_ref,
                 su_ref, tit_ref, shi_ref,
                 *, bu, bi_sz, nbu, nbi, leaky):
    p = pl.program_id(0)
    bi = pl.program_id(1)
    bj = pl.program_id(2)

    first = (bi == 0) & (bj == 0)
    last = (bi == nbu - 1) & (bj == nbi - 1)
    nch = bu // _CH

    @pl.when((p == 0) & first)
    def _init():
        su_ref[...] = jnp.zeros_like(su_ref)
        tit_ref[...] = jnp.zeros_like(tit_ref)
        shi_ref[...] = jnp.zeros_like(shi_ref)


    @pl.when(p == 0)
    def _phase0():
        for k in range(nch):
            sl = pl.ds(k * _CH, _CH)
            su_ref[:, pl.ds(bj * bi_sz, bi_sz)] += jax.lax.dot_general(
                ut_ref[:, pl.ds(bi * bu + k * _CH, _CH)].astype(jnp.bfloat16),
                adj_ref[sl, :].astype(jnp.bfloat16), _MM, preferred_element_type=jnp.float32)
        contrib = jax.lax.dot_general(
            adj_ref[:, pl.ds(0, _CH)].astype(jnp.bfloat16),
            i_ref[pl.ds(0, _CH), :].astype(jnp.bfloat16),
            _MM, preferred_element_type=jnp.float32)
        for k in range(1, bi_sz // _CH):
            contrib += jax.lax.dot_general(
                adj_ref[:, pl.ds(k * _CH, _CH)].astype(jnp.bfloat16),
                i_ref[pl.ds(k * _CH, _CH), :].astype(jnp.bfloat16),
                _MM, preferred_element_type=jnp.float32)
        tit_ref[:, pl.ds(bi * bu, bu)] += contrib.T

    @pl.when(p == 1)
    def _phase1():
        hu = jax.lax.dot_general(
            adj_ref[:, pl.ds(0, _CH)].astype(jnp.bfloat16),
            su_ref[:, pl.ds(bj * bi_sz, _CH)].astype(jnp.bfloat16).T,
            _MM, preferred_element_type=jnp.float32)
        for k in range(1, bi_sz // _CH):
            hu += jax.lax.dot_general(
                adj_ref[:, pl.ds(k * _CH, _CH)].astype(jnp.bfloat16),
                su_ref[:, pl.ds(bj * bi_sz + k * _CH, _CH)]
                .astype(jnp.bfloat16).T,
                _MM, preferred_element_type=jnp.float32)

        @pl.when(bj == 0)
        def _():
            hu_ref[...] = hu

        @pl.when(bj != 0)
        def _():
            hu_ref[...] += hu

        for k in range(nch):
            sl = pl.ds(k * _CH, _CH)
            shi_ref[:, pl.ds(bj * bi_sz, bi_sz)] += jax.lax.dot_general(
                tit_ref[:, pl.ds(bi * bu + k * _CH, _CH)].astype(jnp.bfloat16),
                adj_ref[sl, :].astype(jnp.bfloat16), _MM, preferred_element_type=jnp.float32)

        @pl.when(bj == nbi - 1)
        def _act_u():
            huv = hu_ref[...]
            hu_ref[...] = jnp.where(huv >= 0, huv, leaky * huv)

    @pl.when((p == 1) & last)
    def _act_i():
        hi = shi_ref[...]
        hit_ref[...] = jnp.where(hi >= 0, hi, leaky * hi)


@jax.jit
def kernel(adj, user_emb, item_emb):
    n_users, n_items = adj.shape
    hd = user_emb.shape[1]
    bu = min(n_users, 2048)
    bi_sz = min(n_items, 2048)
    nbu = n_users // bu
    nbi = n_items // bi_sz

    body = functools.partial(_dhcf_kernel, bu=bu, bi_sz=bi_sz,
                             nbu=nbu, nbi=nbi, leaky=0.5)
    h_u, h_i_t = pl.pallas_call(
        body,
        grid=(2, nbu, nbi),
        in_specs=[
            pl.BlockSpec((bu, bi_sz), lambda p, i, j: (i, j)),
            pl.BlockSpec((hd, n_users), lambda p, i, j: (0, 0)),
            pl.BlockSpec((bi_sz, hd), lambda p, i, j: (j, 0)),
        ],
        out_specs=[
            pl.BlockSpec((bu, hd), lambda p, i, j: (i, 0)),
            pl.BlockSpec((hd, n_items), lambda p, i, j: (0, 0)),
        ],
        out_shape=[
            jax.ShapeDtypeStruct((n_users, hd), jnp.float32),
            jax.ShapeDtypeStruct((hd, n_items), jnp.float32),
        ],
        scratch_shapes=[
            pltpu.VMEM((hd, n_items), jnp.float32),
            pltpu.VMEM((hd, n_users), jnp.float32),
            pltpu.VMEM((hd, n_items), jnp.float32),
        ],
    )(adj, user_emb.T, item_emb)

    user_all = jnp.concatenate([user_emb, h_u, h_u], axis=1)
    h_i = h_i_t.T
    item_all = jnp.concatenate([item_emb, h_i, h_i], axis=1)
    return (user_all, item_all)
